# baseline (device time: 187214 ns/iter reference)
import functools

import jax
import jax.numpy as jnp
from jax import lax
from jax.experimental import pallas as pl
from jax.experimental.pallas import tpu as pltpu

N_DEV = 8


def kernel(x, router_W, route_idx, expert_W, shared_W):
    n_per, d = x.shape
    n_exp_local, _, h = expert_W.shape
    n_exp = router_W.shape[1]

    def body(x_ref, rw_ref, idx_ref, ew_ref, sw_ref, out_ref,
             xall, idxall, acc, comm,
             xsend, xrecv, isend, irecv, rs_send, rs_recv):
        my = lax.axis_index("i")
        left = jnp.mod(my - 1, N_DEV)
        right = jnp.mod(my + 1, N_DEV)

        barrier = pltpu.get_barrier_semaphore()
        for nbr in (left, right):
            pl.semaphore_signal(barrier, inc=1, device_id=(nbr,),
                                device_id_type=pl.DeviceIdType.MESH)
        pl.semaphore_wait(barrier, 2)

        xall[my] = x_ref[...]
        idxall[my] = idx_ref[...]

        for k in range(N_DEV - 1):
            o = jnp.mod(my - k, N_DEV)
            cx = pltpu.make_async_remote_copy(
                src_ref=xall.at[o], dst_ref=xall.at[o],
                send_sem=xsend.at[k], recv_sem=xrecv.at[k],
                device_id=(right,), device_id_type=pl.DeviceIdType.MESH)
            ci = pltpu.make_async_remote_copy(
                src_ref=idxall.at[o], dst_ref=idxall.at[o],
                send_sem=isend.at[k], recv_sem=irecv.at[k],
                device_id=(right,), device_id_type=pl.DeviceIdType.MESH)
            cx.start()
            ci.start()
            cx.wait()
            ci.wait()

        rw = rw_ref[...]
        e0 = my * n_exp_local
        cols = lax.broadcasted_iota(jnp.int32, (n_per, n_exp), 1)
        for c in range(N_DEV):
            xc = xall[c]
            idxc = idxall[c]
            scores = jnp.dot(xc, rw, preferred_element_type=jnp.float32)
            smax = jnp.max(scores, axis=-1, keepdims=True)
            ex = jnp.exp(scores - smax)
            denom = jnp.sum(ex, axis=-1, keepdims=True)
            pe = jnp.sum(jnp.where(cols == idxc, ex, 0.0), axis=-1,
                         keepdims=True) / denom
            accv = jnp.zeros((n_per, h), jnp.float32)
            for j in range(n_exp_local):
                coef = jnp.where(idxc == e0 + j, pe, 0.0)
                accv = accv + jnp.dot(xc * coef, ew_ref[j],
                                      preferred_element_type=jnp.float32)
            acc[c] = accv

        for s in range(N_DEV - 1):
            c_send = jnp.mod(my - s - 1, N_DEV)
            slot = s % 2
            r = pltpu.make_async_remote_copy(
                src_ref=acc.at[c_send], dst_ref=comm.at[slot],
                send_sem=rs_send.at[s], recv_sem=rs_recv.at[s],
                device_id=(right,), device_id_type=pl.DeviceIdType.MESH)
            r.start()
            r.wait()
            c_recv = jnp.mod(my - s - 2, N_DEV)
            acc[c_recv] = acc[c_recv] + comm[slot]

        shared = jnp.dot(x_ref[...], sw_ref[...],
                         preferred_element_type=jnp.float32)
        out_ref[...] = acc[my] + shared

        @functools.partial(pl.run_scoped,
                           second_barrier=pltpu.SemaphoreType.REGULAR)
        def _(second_barrier):
            for nbr in (left, right):
                pl.semaphore_signal(second_barrier, inc=1, device_id=(nbr,),
                                    device_id_type=pl.DeviceIdType.MESH)
            pl.semaphore_wait(second_barrier, 2)

    return pl.pallas_call(
        body,
        out_shape=jax.ShapeDtypeStruct((n_per, h), jnp.float32),
        in_specs=[pl.BlockSpec(memory_space=pltpu.VMEM)] * 5,
        out_specs=pl.BlockSpec(memory_space=pltpu.VMEM),
        scratch_shapes=[
            pltpu.VMEM((N_DEV, n_per, d), jnp.float32),
            pltpu.VMEM((N_DEV, n_per, 1), jnp.int32),
            pltpu.VMEM((N_DEV, n_per, h), jnp.float32),
            pltpu.VMEM((2, n_per, h), jnp.float32),
            pltpu.SemaphoreType.DMA((N_DEV - 1,)),
            pltpu.SemaphoreType.DMA((N_DEV - 1,)),
            pltpu.SemaphoreType.DMA((N_DEV - 1,)),
            pltpu.SemaphoreType.DMA((N_DEV - 1,)),
            pltpu.SemaphoreType.DMA((N_DEV - 1,)),
            pltpu.SemaphoreType.DMA((N_DEV - 1,)),
        ],
        compiler_params=pltpu.CompilerParams(collective_id=0),
    )(x, router_W, route_idx, expert_W, shared_W)


# device time: 154900 ns/iter; 1.2086x vs baseline; 1.2086x over previous
import functools

import jax
import jax.numpy as jnp
from jax import lax
from jax.experimental import pallas as pl
from jax.experimental.pallas import tpu as pltpu

N_DEV = 8


def kernel(x, router_W, route_idx, expert_W, shared_W):
    n_per, d = x.shape
    n_exp_local, _, h = expert_W.shape
    n_exp = router_W.shape[1]

    def body(x_ref, rw_ref, idx_ref, ew_ref, sw_ref, out_ref,
             xall, idxall, acc, comm,
             xsend, xrecv, isend, irecv, rs_send, rs_recv):
        my = lax.axis_index("i")
        left = jnp.mod(my - 1, N_DEV)
        right = jnp.mod(my + 1, N_DEV)

        barrier = pltpu.get_barrier_semaphore()
        for nbr in (left, right):
            pl.semaphore_signal(barrier, inc=1, device_id=(nbr,),
                                device_id_type=pl.DeviceIdType.MESH)
        pl.semaphore_wait(barrier, 2)

        xall[my] = x_ref[...]
        idxall[my] = idx_ref[...]

        rw = rw_ref[...]
        e0 = my * n_exp_local
        cols = lax.broadcasted_iota(jnp.int32, (n_per, n_exp), 1)

        def compute_chunk(c):
            xc = xall[c]
            idxc = idxall[c]
            scores = jnp.dot(xc, rw, preferred_element_type=jnp.float32)
            smax = jnp.max(scores, axis=-1, keepdims=True)
            ex = jnp.exp(scores - smax)
            denom = jnp.sum(ex, axis=-1, keepdims=True)
            pe = jnp.sum(jnp.where(cols == idxc, ex, 0.0), axis=-1,
                         keepdims=True) / denom
            accv = jnp.zeros((n_per, h), jnp.float32)
            for j in range(n_exp_local):
                coef = jnp.where(idxc == e0 + j, pe, 0.0)
                accv = accv + jnp.dot(xc * coef, ew_ref[j],
                                      preferred_element_type=jnp.float32)
            acc[c] = accv

        ag = []
        for k in range(N_DEV - 1):
            o = jnp.mod(my - k, N_DEV)
            ag.append((
                pltpu.make_async_remote_copy(
                    src_ref=xall.at[o], dst_ref=xall.at[o],
                    send_sem=xsend.at[k], recv_sem=xrecv.at[k],
                    device_id=(right,), device_id_type=pl.DeviceIdType.MESH),
                pltpu.make_async_remote_copy(
                    src_ref=idxall.at[o], dst_ref=idxall.at[o],
                    send_sem=isend.at[k], recv_sem=irecv.at[k],
                    device_id=(right,), device_id_type=pl.DeviceIdType.MESH),
            ))
        rs = []
        for s in range(N_DEV - 1):
            c_s = jnp.mod(my - s - 1, N_DEV)
            rs.append(pltpu.make_async_remote_copy(
                src_ref=acc.at[c_s], dst_ref=comm.at[s],
                send_sem=rs_send.at[s], recv_sem=rs_recv.at[s],
                device_id=(right,), device_id_type=pl.DeviceIdType.MESH))

        ag[0][0].start()
        ag[0][1].start()
        compute_chunk(my)
        for s in range(N_DEV - 1):
            ag[s][0].wait_recv()
            ag[s][1].wait_recv()
            if s + 1 < N_DEV - 1:
                ag[s + 1][0].start()
                ag[s + 1][1].start()
            c = jnp.mod(my - s - 1, N_DEV)
            compute_chunk(c)
            if s > 0:
                rs[s - 1].wait_recv()
                acc[c] = acc[c] + comm[s - 1]
            rs[s].start()
        rs[N_DEV - 2].wait_recv()
        acc[my] = acc[my] + comm[N_DEV - 2]

        shared = jnp.dot(x_ref[...], sw_ref[...],
                         preferred_element_type=jnp.float32)
        out_ref[...] = acc[my] + shared

        for s in range(N_DEV - 1):
            ag[s][0].wait_send()
            ag[s][1].wait_send()
            rs[s].wait_send()

        @functools.partial(pl.run_scoped,
                           second_barrier=pltpu.SemaphoreType.REGULAR)
        def _(second_barrier):
            for nbr in (left, right):
                pl.semaphore_signal(second_barrier, inc=1, device_id=(nbr,),
                                    device_id_type=pl.DeviceIdType.MESH)
            pl.semaphore_wait(second_barrier, 2)

    return pl.pallas_call(
        body,
        out_shape=jax.ShapeDtypeStruct((n_per, h), jnp.float32),
        in_specs=[pl.BlockSpec(memory_space=pltpu.VMEM)] * 5,
        out_specs=pl.BlockSpec(memory_space=pltpu.VMEM),
        scratch_shapes=[
            pltpu.VMEM((N_DEV, n_per, d), jnp.float32),
            pltpu.VMEM((N_DEV, n_per, 1), jnp.int32),
            pltpu.VMEM((N_DEV, n_per, h), jnp.float32),
            pltpu.VMEM((N_DEV - 1, n_per, h), jnp.float32),
            pltpu.SemaphoreType.DMA((N_DEV - 1,)),
            pltpu.SemaphoreType.DMA((N_DEV - 1,)),
            pltpu.SemaphoreType.DMA((N_DEV - 1,)),
            pltpu.SemaphoreType.DMA((N_DEV - 1,)),
            pltpu.SemaphoreType.DMA((N_DEV - 1,)),
            pltpu.SemaphoreType.DMA((N_DEV - 1,)),
        ],
        compiler_params=pltpu.CompilerParams(collective_id=0),
    )(x, router_W, route_idx, expert_W, shared_W)


# device time: 96070 ns/iter; 1.9487x vs baseline; 1.6124x over previous
import functools

import jax
import jax.numpy as jnp
from jax import lax
from jax.experimental import pallas as pl
from jax.experimental.pallas import tpu as pltpu

N_DEV = 8


def kernel(x, router_W, route_idx, expert_W, shared_W):
    n_per, d = x.shape
    n_exp_local, _, h = expert_W.shape
    n_exp = router_W.shape[1]

    def body(x_ref, rw_ref, idx_ref, ew_ref, sw_ref, out_ref,
             xall, idxall, acc, accb, comm,
             xsend, xrecv, isend, irecv, rs_send, rs_recv):
        my = lax.axis_index("i")
        left = jnp.mod(my - 1, N_DEV)
        right = jnp.mod(my + 1, N_DEV)

        barrier = pltpu.get_barrier_semaphore()
        for nbr in (left, right):
            pl.semaphore_signal(barrier, inc=1, device_id=(nbr,),
                                device_id_type=pl.DeviceIdType.MESH)
        pl.semaphore_wait(barrier, 2)

        xall[my] = x_ref[...].astype(jnp.bfloat16)
        idxall[my] = idx_ref[...]

        rw = rw_ref[...]
        ewb = ew_ref[...].astype(jnp.bfloat16)
        e0 = my * n_exp_local
        cols = lax.broadcasted_iota(jnp.int32, (n_per, n_exp), 1)

        def compute_chunk(c):
            xc = xall[c].astype(jnp.float32)
            idxc = idxall[c]
            scores = jnp.dot(xc, rw, preferred_element_type=jnp.float32)
            smax = jnp.max(scores, axis=-1, keepdims=True)
            ex = jnp.exp(scores - smax)
            denom = jnp.sum(ex, axis=-1, keepdims=True)
            pe = jnp.sum(jnp.where(cols == idxc, ex, 0.0), axis=-1,
                         keepdims=True) / denom
            accv = jnp.zeros((n_per, h), jnp.float32)
            for j in range(n_exp_local):
                coef = jnp.where(idxc == e0 + j, pe, 0.0)
                accv = accv + jnp.dot((xc * coef).astype(jnp.bfloat16),
                                      ewb[j],
                                      preferred_element_type=jnp.float32)
            acc[c] = accv
            accb[c] = accv.astype(jnp.bfloat16)

        ag = []
        for k in range(N_DEV - 1):
            o = jnp.mod(my - k, N_DEV)
            ag.append((
                pltpu.make_async_remote_copy(
                    src_ref=xall.at[o], dst_ref=xall.at[o],
                    send_sem=xsend.at[k], recv_sem=xrecv.at[k],
                    device_id=(right,), device_id_type=pl.DeviceIdType.MESH),
                pltpu.make_async_remote_copy(
                    src_ref=idxall.at[o], dst_ref=idxall.at[o],
                    send_sem=isend.at[k], recv_sem=irecv.at[k],
                    device_id=(right,), device_id_type=pl.DeviceIdType.MESH),
            ))
        rs = []
        for s in range(N_DEV - 1):
            c_s = jnp.mod(my - s - 1, N_DEV)
            rs.append(pltpu.make_async_remote_copy(
                src_ref=accb.at[c_s], dst_ref=comm.at[s],
                send_sem=rs_send.at[s], recv_sem=rs_recv.at[s],
                device_id=(right,), device_id_type=pl.DeviceIdType.MESH))

        ag[0][0].start()
        ag[0][1].start()
        compute_chunk(my)
        for s in range(N_DEV - 1):
            ag[s][0].wait_recv()
            ag[s][1].wait_recv()
            if s + 1 < N_DEV - 1:
                ag[s + 1][0].start()
                ag[s + 1][1].start()
            c = jnp.mod(my - s - 1, N_DEV)
            compute_chunk(c)
            if s > 0:
                rs[s - 1].wait_recv()
                accv = acc[c] + comm[s - 1].astype(jnp.float32)
                acc[c] = accv
                accb[c] = accv.astype(jnp.bfloat16)
            rs[s].start()
        rs[N_DEV - 2].wait_recv()
        acc[my] = acc[my] + comm[N_DEV - 2].astype(jnp.float32)

        shared = jnp.dot(x_ref[...], sw_ref[...],
                         preferred_element_type=jnp.float32)
        out_ref[...] = acc[my] + shared

        for s in range(N_DEV - 1):
            ag[s][0].wait_send()
            ag[s][1].wait_send()
            rs[s].wait_send()

        @functools.partial(pl.run_scoped,
                           second_barrier=pltpu.SemaphoreType.REGULAR)
        def _(second_barrier):
            for nbr in (left, right):
                pl.semaphore_signal(second_barrier, inc=1, device_id=(nbr,),
                                    device_id_type=pl.DeviceIdType.MESH)
            pl.semaphore_wait(second_barrier, 2)

    return pl.pallas_call(
        body,
        out_shape=jax.ShapeDtypeStruct((n_per, h), jnp.float32),
        in_specs=[pl.BlockSpec(memory_space=pltpu.VMEM)] * 5,
        out_specs=pl.BlockSpec(memory_space=pltpu.VMEM),
        scratch_shapes=[
            pltpu.VMEM((N_DEV, n_per, d), jnp.bfloat16),
            pltpu.VMEM((N_DEV, n_per, 1), jnp.int32),
            pltpu.VMEM((N_DEV, n_per, h), jnp.float32),
            pltpu.VMEM((N_DEV, n_per, h), jnp.bfloat16),
            pltpu.VMEM((N_DEV - 1, n_per, h), jnp.bfloat16),
            pltpu.SemaphoreType.DMA((N_DEV - 1,)),
            pltpu.SemaphoreType.DMA((N_DEV - 1,)),
            pltpu.SemaphoreType.DMA((N_DEV - 1,)),
            pltpu.SemaphoreType.DMA((N_DEV - 1,)),
            pltpu.SemaphoreType.DMA((N_DEV - 1,)),
            pltpu.SemaphoreType.DMA((N_DEV - 1,)),
        ],
        compiler_params=pltpu.CompilerParams(collective_id=0),
    )(x, router_W, route_idx, expert_W, shared_W)
